# Initial kernel scaffold; baseline (speedup 1.0000x reference)
import functools
import jax
import jax.numpy as jnp
from jax import lax
from jax.experimental import pallas as pl
from jax.experimental.pallas import tpu as pltpu
from jax.experimental.pallas import tpu_sc as plsc

NC = 2
NS = 16
CH = 128
RB = 1024


def _mesh():
    return plsc.VectorSubcoreMesh(core_axis_name="c", subcore_axis_name="s",
                                  num_cores=NC, num_subcores=NS)


def _mm_body(x_ref, w_ref, o_ref):
    o_ref[...] = jnp.dot(x_ref[...], w_ref[...],
                         preferred_element_type=jnp.float32)


def _tc_matmul(x, w, np_rows):
    return pl.pallas_call(
        _mm_body,
        grid=(np_rows // RB,),
        in_specs=[pl.BlockSpec((RB, x.shape[1]), lambda i: (i, 0)),
                  pl.BlockSpec(w.shape, lambda i: (0, 0))],
        out_specs=pl.BlockSpec((RB, w.shape[1]), lambda i: (i, 0)),
        out_shape=jax.ShapeDtypeStruct((np_rows, w.shape[1]), jnp.float32),
    )(x, w)


def _k3_body(h1p_ref, d0_ref, d1_ref, dinv_ref, glo_ref, ghi_ref):
    deg = d0_ref[:, 0:1] + d1_ref[:, 0:1] + 1.0
    dinv = lax.rsqrt(deg)
    dinv_ref[...] = dinv
    g = h1p_ref[...] * dinv
    glo_ref[...] = g[:, :64]
    ghi_ref[...] = g[:, 64:]


def _k3(h1p, deg_st, np_rows):
    return pl.pallas_call(
        _k3_body,
        grid=(np_rows // RB,),
        in_specs=[pl.BlockSpec((RB, 128), lambda i: (i, 0)),
                  pl.BlockSpec((1, RB, 16), lambda i: (0, i, 0)),
                  pl.BlockSpec((1, RB, 16), lambda i: (1, i, 0))],
        out_specs=[pl.BlockSpec((RB, 1), lambda i: (i, 0)),
                   pl.BlockSpec((RB, 64), lambda i: (i, 0)),
                   pl.BlockSpec((RB, 64), lambda i: (i, 0))],
        out_shape=[jax.ShapeDtypeStruct((np_rows, 1), jnp.float32),
                   jax.ShapeDtypeStruct((np_rows, 64), jnp.float32),
                   jax.ShapeDtypeStruct((np_rows, 64), jnp.float32)],
    )(h1p, deg_st, deg_st)


def _k5_body(alo_ref, ahi_ref, glo_ref, ghi_ref, dinv_ref, b_ref, w_ref,
             olo_ref, ohi_ref):
    acc = jnp.concatenate([alo_ref[...], ahi_ref[...]], axis=1)
    g = jnp.concatenate([glo_ref[...], ghi_ref[...]], axis=1)
    dinv = dinv_ref[...]
    h = jnp.maximum(dinv * (acc + g) + b_ref[...], 0.0)
    h2p = jnp.dot(h, w_ref[...], preferred_element_type=jnp.float32)
    g2 = h2p * dinv
    olo_ref[...] = g2[:, :64]
    ohi_ref[...] = g2[:, 64:]


def _k5(alo, ahi, glo, ghi, dinv, b, w, np_rows):
    return pl.pallas_call(
        _k5_body,
        grid=(np_rows // RB,),
        in_specs=[pl.BlockSpec((RB, 64), lambda i: (i, 0)),
                  pl.BlockSpec((RB, 64), lambda i: (i, 0)),
                  pl.BlockSpec((RB, 64), lambda i: (i, 0)),
                  pl.BlockSpec((RB, 64), lambda i: (i, 0)),
                  pl.BlockSpec((RB, 1), lambda i: (i, 0)),
                  pl.BlockSpec((1, 128), lambda i: (0, 0)),
                  pl.BlockSpec((128, 128), lambda i: (0, 0))],
        out_specs=[pl.BlockSpec((RB, 64), lambda i: (i, 0)),
                   pl.BlockSpec((RB, 64), lambda i: (i, 0))],
        out_shape=[jax.ShapeDtypeStruct((np_rows, 64), jnp.float32),
                   jax.ShapeDtypeStruct((np_rows, 64), jnp.float32)],
    )(alo, ahi, glo, ghi, dinv, b, w)


def _k7_body(alo_ref, ahi_ref, glo_ref, ghi_ref, dinv_ref, b_ref, o_ref):
    acc = jnp.concatenate([alo_ref[...], ahi_ref[...]], axis=1)
    g = jnp.concatenate([glo_ref[...], ghi_ref[...]], axis=1)
    o_ref[...] = jnp.maximum(dinv_ref[...] * (acc + g) + b_ref[...], 0.0)


def _k7(alo, ahi, glo, ghi, dinv, b, np_rows):
    return pl.pallas_call(
        _k7_body,
        grid=(np_rows // RB,),
        in_specs=[pl.BlockSpec((RB, 64), lambda i: (i, 0)),
                  pl.BlockSpec((RB, 64), lambda i: (i, 0)),
                  pl.BlockSpec((RB, 64), lambda i: (i, 0)),
                  pl.BlockSpec((RB, 64), lambda i: (i, 0)),
                  pl.BlockSpec((RB, 1), lambda i: (i, 0)),
                  pl.BlockSpec((1, 128), lambda i: (0, 0))],
        out_specs=pl.BlockSpec((RB, 128), lambda i: (i, 0)),
        out_shape=jax.ShapeDtypeStruct((np_rows, 128), jnp.float32),
    )(alo, ahi, glo, ghi, dinv, b)


def _k9_body(s_ref, c0_ref, c1_ref, wc_ref, bc_ref, o_ref):
    sums = s_ref[0] + s_ref[1]
    cnt = c0_ref[0, :, 0:1] + c1_ref[0, :, 0:1]
    pooled = sums / jnp.clip(cnt, 1.0, None)
    o_ref[...] = jnp.dot(pooled, wc_ref[...],
                         preferred_element_type=jnp.float32) + bc_ref[...]


def _k9(sums_st, cnt_st, wc, bc):
    return pl.pallas_call(
        _k9_body,
        grid=(1,),
        in_specs=[pl.BlockSpec((2, 128, 128), lambda i: (0, 0, 0)),
                  pl.BlockSpec((1, 128, 16), lambda i: (0, 0, 0)),
                  pl.BlockSpec((1, 128, 16), lambda i: (1, 0, 0)),
                  pl.BlockSpec((128, 1), lambda i: (0, 0)),
                  pl.BlockSpec((1, 1), lambda i: (0, 0))],
        out_specs=pl.BlockSpec((128, 1), lambda i: (0, 0)),
        out_shape=jax.ShapeDtypeStruct((128, 1), jnp.float32),
    )(sums_st, cnt_st, cnt_st, wc, bc)


def _sc_deg(dst_pad, batch_pad, np_rows, ep, nb):
    rows_per_tile = np_rows // (NC * NS)
    nchunk = ep // (CH * NC * NS)
    bch = 64
    nbchunk = rows_per_tile // bch
    cnt_rows = 2 * nb

    @functools.partial(
        pl.kernel,
        out_type=[jax.ShapeDtypeStruct((NC, np_rows, 16), jnp.float32),
                  jax.ShapeDtypeStruct((NC, nb, 16), jnp.float32)],
        mesh=_mesh(),
        scratch_types=[
            pltpu.VMEM((CH,), jnp.int32),
            pltpu.VMEM((64,), jnp.int32),
            pltpu.VMEM((CH, 16), jnp.float32),
            pltpu.VMEM((CH, 16), jnp.float32),
            pltpu.VMEM_SHARED((np_rows, 16), jnp.float32),
            pltpu.VMEM_SHARED((2 * nb, 16), jnp.float32),
        ],
    )
    def k(dst_hbm, batch_hbm, deg_out, cnt_out, idx_v, bidx_v, ones_v, zbuf_v,
          acc_sh, cnt_sh):
        c = lax.axis_index("c")
        s = lax.axis_index("s")
        w = c * NS + s

        def fill(i, _):
            ones_v[i, :] = jnp.full((16,), 1.0, jnp.float32)
            zbuf_v[i, :] = jnp.zeros((16,), jnp.float32)
            return 0
        lax.fori_loop(0, CH, fill, 0)

        def zrow(j, _):
            pltpu.sync_copy(zbuf_v, acc_sh.at[pl.ds(s * rows_per_tile
                                                    + j * CH, CH)])
            return 0
        lax.fori_loop(0, rows_per_tile // CH, zrow, 0)

        @pl.when(s == 0)
        def _():
            pltpu.sync_copy(zbuf_v.at[pl.ds(0, cnt_rows)],
                            cnt_sh.at[pl.ds(0, cnt_rows)])

        plsc.subcore_barrier()

        def body(j, _):
            base = (w * nchunk + j) * CH
            pltpu.sync_copy(dst_hbm.at[pl.ds(base, CH)], idx_v)
            pltpu.sync_copy(ones_v, acc_sh.at[idx_v], add=True)
            return 0
        lax.fori_loop(0, nchunk, body, 0)

        def bbody(j, _):
            base = (w * nbchunk + j) * bch
            pltpu.sync_copy(batch_hbm.at[pl.ds(base, bch)], bidx_v)
            pltpu.sync_copy(ones_v.at[pl.ds(0, bch)], cnt_sh.at[bidx_v],
                            add=True)
            return 0
        lax.fori_loop(0, nbchunk, bbody, 0)

        plsc.subcore_barrier()

        pltpu.sync_copy(acc_sh.at[pl.ds(s * rows_per_tile, rows_per_tile)],
                        deg_out.at[c, pl.ds(s * rows_per_tile, rows_per_tile)])

        @pl.when(s == 0)
        def _():
            pltpu.sync_copy(cnt_sh.at[pl.ds(0, nb)], cnt_out.at[c])

    return k(dst_pad, batch_pad)


def _sc_agg(glo, ghi, src_pad, dst_pad, np_rows, ep):
    rows_per_tile = np_rows // NS
    nchunk = ep // (CH * NS)

    @functools.partial(
        pl.kernel,
        out_type=[jax.ShapeDtypeStruct((np_rows, 64), jnp.float32),
                  jax.ShapeDtypeStruct((np_rows, 64), jnp.float32)],
        mesh=_mesh(),
        scratch_types=[
            pltpu.VMEM((CH,), jnp.int32),
            pltpu.VMEM((CH,), jnp.int32),
            pltpu.VMEM((CH, 64), jnp.float32),
            pltpu.VMEM((CH, 64), jnp.float32),
            pltpu.VMEM_SHARED((np_rows, 64), jnp.float32),
            pltpu.SemaphoreType.DMA,
        ],
    )
    def k(glo_hbm, ghi_hbm, src_hbm, dst_hbm, out_lo, out_hi, sidx_v, didx_v,
          rows_v, zbuf_v, acc_sh, gsem):
        c = lax.axis_index("c")
        s = lax.axis_index("s")

        def zf(i, _):
            r = i // 4
            q = i % 4
            zbuf_v[r, pl.ds(q * 16, 16)] = jnp.zeros((16,), jnp.float32)
            return 0
        lax.fori_loop(0, CH * 4, zf, 0)

        def zrow(j, _):
            pltpu.sync_copy(zbuf_v, acc_sh.at[pl.ds(s * rows_per_tile
                                                    + j * CH, CH)])
            return 0
        lax.fori_loop(0, rows_per_tile // CH, zrow, 0)
        plsc.subcore_barrier()

        def run(tbl_hbm, out_hbm):
            def body(j, _):
                base = (s * nchunk + j) * CH
                pltpu.sync_copy(src_hbm.at[pl.ds(base, CH)], sidx_v)
                pltpu.sync_copy(dst_hbm.at[pl.ds(base, CH)], didx_v)
                pltpu.async_copy(tbl_hbm.at[sidx_v], rows_v, gsem).wait()
                pltpu.sync_copy(rows_v, acc_sh.at[didx_v], add=True)
                return 0
            lax.fori_loop(0, nchunk, body, 0)
            plsc.subcore_barrier()
            pltpu.sync_copy(
                acc_sh.at[pl.ds(s * rows_per_tile, rows_per_tile)],
                out_hbm.at[pl.ds(s * rows_per_tile, rows_per_tile)])

        @pl.when(c == 0)
        def _():
            run(glo_hbm, out_lo)

        @pl.when(c == 1)
        def _():
            run(ghi_hbm, out_hi)

    return k(glo, ghi, src_pad, dst_pad)


def _sc_pool(h2, batch_pad, np_rows, nb):
    bch = 64
    rows_per_tile = np_rows // (NC * NS)
    nchunk = rows_per_tile // bch
    acc_rows = 2 * nb

    @functools.partial(
        pl.kernel,
        out_type=jax.ShapeDtypeStruct((NC, nb, 128), jnp.float32),
        mesh=_mesh(),
        scratch_types=[
            pltpu.VMEM((bch,), jnp.int32),
            pltpu.VMEM((bch, 128), jnp.float32),
            pltpu.VMEM((CH, 128), jnp.float32),
            pltpu.VMEM_SHARED((2 * nb, 128), jnp.float32),
        ],
    )
    def k(h2_hbm, batch_hbm, out_hbm, bidx_v, rows_v, zbuf_v, acc_sh):
        c = lax.axis_index("c")
        s = lax.axis_index("s")
        w = c * NS + s

        def zf(i, _):
            r = i // 8
            q = i % 8
            zbuf_v[r, pl.ds(q * 16, 16)] = jnp.zeros((16,), jnp.float32)
            return 0
        lax.fori_loop(0, CH * 8, zf, 0)

        @pl.when(s == 0)
        def _():
            pltpu.sync_copy(zbuf_v.at[pl.ds(0, acc_rows)],
                            acc_sh.at[pl.ds(0, acc_rows)])
        plsc.subcore_barrier()

        def body(j, _):
            base = (w * nchunk + j) * bch
            pltpu.sync_copy(batch_hbm.at[pl.ds(base, bch)], bidx_v)
            pltpu.sync_copy(h2_hbm.at[pl.ds(base, bch)], rows_v)
            pltpu.sync_copy(rows_v, acc_sh.at[bidx_v], add=True)
            return 0
        lax.fori_loop(0, nchunk, body, 0)
        plsc.subcore_barrier()

        @pl.when(s < 8)
        def _():
            pltpu.sync_copy(acc_sh.at[pl.ds(s * 16, 16)],
                            out_hbm.at[c, pl.ds(s * 16, 16)])

    return k(h2, batch_pad)


def kernel(x, edge_index, batch, W1, b1, W2, b2, Wc, bc):
    n, d = x.shape
    e = edge_index.shape[1]
    nb = 128

    unit = CH * NC * NS
    np_rows = ((n + unit - 1) // unit) * unit
    ep = ((e + unit - 1) // unit) * unit

    x_pad = jnp.pad(x, ((0, np_rows - n), (0, 0)))
    pad_idx = jnp.full((ep - e,), n, jnp.int32)
    src = jnp.concatenate([edge_index[0], pad_idx])
    dst = jnp.concatenate([edge_index[1], pad_idx])
    batch_pad = jnp.concatenate(
        [batch, jnp.full((np_rows - n,), nb, jnp.int32)])

    b1r = b1.reshape(1, 128)
    b2r = b2.reshape(1, 128)
    bcr = bc.reshape(1, 1)

    h1p = _tc_matmul(x_pad, W1, np_rows)
    deg_st, cnt_st = _sc_deg(dst, batch_pad, np_rows, ep, nb)
    dinv, g1lo, g1hi = _k3(h1p, deg_st, np_rows)
    a1lo, a1hi = _sc_agg(g1lo, g1hi, src, dst, np_rows, ep)
    g2lo, g2hi = _k5(a1lo, a1hi, g1lo, g1hi, dinv, b1r, W2, np_rows)
    a2lo, a2hi = _sc_agg(g2lo, g2hi, src, dst, np_rows, ep)
    h2 = _k7(a2lo, a2hi, g2lo, g2hi, dinv, b2r, np_rows)
    sums_st = _sc_pool(h2, batch_pad, np_rows, nb)
    return _k9(sums_st, cnt_st, Wc, bcr)


# speedup vs baseline: 10.6056x; 10.6056x over previous
import functools
import jax
import jax.numpy as jnp
from jax import lax
from jax.experimental import pallas as pl
from jax.experimental.pallas import tpu as pltpu
from jax.experimental.pallas import tpu_sc as plsc

NC = 2
NS = 16
CH = 128
RB = 1024


def _mesh():
    return plsc.VectorSubcoreMesh(core_axis_name="c", subcore_axis_name="s",
                                  num_cores=NC, num_subcores=NS)


def _mm_body(x_ref, w_ref, o_ref):
    o_ref[...] = jnp.dot(x_ref[...], w_ref[...],
                         preferred_element_type=jnp.float32)


def _tc_matmul(x, w, np_rows):
    return pl.pallas_call(
        _mm_body,
        grid=(np_rows // RB,),
        in_specs=[pl.BlockSpec((RB, x.shape[1]), lambda i: (i, 0)),
                  pl.BlockSpec(w.shape, lambda i: (0, 0))],
        out_specs=pl.BlockSpec((RB, w.shape[1]), lambda i: (i, 0)),
        out_shape=jax.ShapeDtypeStruct((np_rows, w.shape[1]), jnp.float32),
    )(x, w)


def _k3_body(h1p_ref, d0_ref, d1_ref, dinv_ref, g_ref):
    deg = d0_ref[0, :, 0:1] + d1_ref[0, :, 0:1] + 1.0
    dinv = lax.rsqrt(deg)
    dinv_ref[...] = dinv
    g_ref[...] = h1p_ref[...] * dinv


def _k3(h1p, deg_st, np_rows):
    return pl.pallas_call(
        _k3_body,
        grid=(np_rows // RB,),
        in_specs=[pl.BlockSpec((RB, 128), lambda i: (i, 0)),
                  pl.BlockSpec((1, RB, 128), lambda i: (0, i, 0)),
                  pl.BlockSpec((1, RB, 128), lambda i: (1, i, 0))],
        out_specs=[pl.BlockSpec((RB, 1), lambda i: (i, 0)),
                   pl.BlockSpec((RB, 128), lambda i: (i, 0))],
        out_shape=[jax.ShapeDtypeStruct((np_rows, 1), jnp.float32),
                   jax.ShapeDtypeStruct((np_rows, 128), jnp.float32)],
    )(h1p, deg_st, deg_st)


def _k5_body(a0_ref, a1_ref, g_ref, dinv_ref, b_ref, w_ref, o_ref):
    acc = a0_ref[0] + a1_ref[0]
    dinv = dinv_ref[...]
    h = jnp.maximum(dinv * (acc + g_ref[...]) + b_ref[...], 0.0)
    h2p = jnp.dot(h, w_ref[...], preferred_element_type=jnp.float32)
    o_ref[...] = h2p * dinv


def _k5(a_st, g, dinv, b, w, np_rows):
    return pl.pallas_call(
        _k5_body,
        grid=(np_rows // RB,),
        in_specs=[pl.BlockSpec((1, RB, 128), lambda i: (0, i, 0)),
                  pl.BlockSpec((1, RB, 128), lambda i: (1, i, 0)),
                  pl.BlockSpec((RB, 128), lambda i: (i, 0)),
                  pl.BlockSpec((RB, 1), lambda i: (i, 0)),
                  pl.BlockSpec((1, 128), lambda i: (0, 0)),
                  pl.BlockSpec((128, 128), lambda i: (0, 0))],
        out_specs=pl.BlockSpec((RB, 128), lambda i: (i, 0)),
        out_shape=jax.ShapeDtypeStruct((np_rows, 128), jnp.float32),
    )(a_st, a_st, g, dinv, b, w)


def _k7_body(a0_ref, a1_ref, g_ref, dinv_ref, b_ref, o_ref):
    acc = a0_ref[0] + a1_ref[0]
    o_ref[...] = jnp.maximum(
        dinv_ref[...] * (acc + g_ref[...]) + b_ref[...], 0.0)


def _k7(a_st, g, dinv, b, np_rows):
    return pl.pallas_call(
        _k7_body,
        grid=(np_rows // RB,),
        in_specs=[pl.BlockSpec((1, RB, 128), lambda i: (0, i, 0)),
                  pl.BlockSpec((1, RB, 128), lambda i: (1, i, 0)),
                  pl.BlockSpec((RB, 128), lambda i: (i, 0)),
                  pl.BlockSpec((RB, 1), lambda i: (i, 0)),
                  pl.BlockSpec((1, 128), lambda i: (0, 0))],
        out_specs=pl.BlockSpec((RB, 128), lambda i: (i, 0)),
        out_shape=jax.ShapeDtypeStruct((np_rows, 128), jnp.float32),
    )(a_st, a_st, g, dinv, b)


def _k9_body(s_ref, c0_ref, c1_ref, wc_ref, bc_ref, o_ref):
    sums = s_ref[0] + s_ref[1]
    cnt = c0_ref[0, :, 0:1] + c1_ref[0, :, 0:1]
    pooled = sums / jnp.clip(cnt, 1.0, None)
    o_ref[...] = jnp.dot(pooled, wc_ref[...],
                         preferred_element_type=jnp.float32) + bc_ref[...]


def _k9(sums_st, cnt_st, wc, bc):
    return pl.pallas_call(
        _k9_body,
        grid=(1,),
        in_specs=[pl.BlockSpec((2, 128, 128), lambda i: (0, 0, 0)),
                  pl.BlockSpec((1, 128, 128), lambda i: (0, 0, 0)),
                  pl.BlockSpec((1, 128, 128), lambda i: (1, 0, 0)),
                  pl.BlockSpec((128, 1), lambda i: (0, 0)),
                  pl.BlockSpec((1, 1), lambda i: (0, 0))],
        out_specs=pl.BlockSpec((128, 1), lambda i: (0, 0)),
        out_shape=jax.ShapeDtypeStruct((128, 1), jnp.float32),
    )(sums_st, cnt_st, cnt_st, wc, bc)


def _sc_deg(dst_pad, batch_pad, np_rows, ep, nb):
    rows_per_tile = np_rows // (NC * NS)
    zrows_per_tile = np_rows // NS
    nchunk = ep // (CH * NC * NS)
    bch = 64
    nbchunk = rows_per_tile // bch
    cnt_rows = 2 * nb

    @functools.partial(
        pl.kernel,
        out_type=[jax.ShapeDtypeStruct((NC, np_rows, 128), jnp.float32),
                  jax.ShapeDtypeStruct((NC, nb, 128), jnp.float32)],
        mesh=_mesh(),
        scratch_types=[
            pltpu.VMEM((CH,), jnp.int32),
            pltpu.VMEM((64,), jnp.int32),
            pltpu.VMEM((CH, 128), jnp.float32),
            pltpu.VMEM((CH, 128), jnp.float32),
            pltpu.VMEM_SHARED((np_rows, 128), jnp.float32),
            pltpu.VMEM_SHARED((2 * nb, 128), jnp.float32),
        ],
    )
    def k(dst_hbm, batch_hbm, deg_out, cnt_out, idx_v, bidx_v, ones_v, zbuf_v,
          acc_sh, cnt_sh):
        c = lax.axis_index("c")
        s = lax.axis_index("s")
        w = c * NS + s

        def fill(i, _):
            r = i // 8
            q = i % 8
            ones_v[r, pl.ds(q * 16, 16)] = jnp.full((16,), 1.0, jnp.float32)
            zbuf_v[r, pl.ds(q * 16, 16)] = jnp.zeros((16,), jnp.float32)
            return 0
        lax.fori_loop(0, CH * 8, fill, 0)

        def zrow(j, _):
            pltpu.sync_copy(zbuf_v, acc_sh.at[pl.ds(s * zrows_per_tile
                                                    + j * CH, CH)])
            return 0
        lax.fori_loop(0, zrows_per_tile // CH, zrow, 0)

        @pl.when(s < cnt_rows // CH)
        def _():
            pltpu.sync_copy(zbuf_v, cnt_sh.at[pl.ds(s * CH, CH)])

        plsc.subcore_barrier()

        def body(j, _):
            base = (w * nchunk + j) * CH
            pltpu.sync_copy(dst_hbm.at[pl.ds(base, CH)], idx_v)
            pltpu.sync_copy(ones_v, acc_sh.at[idx_v], add=True)
            return 0
        lax.fori_loop(0, nchunk, body, 0)

        def bbody(j, _):
            base = (w * nbchunk + j) * bch
            pltpu.sync_copy(batch_hbm.at[pl.ds(base, bch)], bidx_v)
            pltpu.sync_copy(ones_v.at[pl.ds(0, bch)], cnt_sh.at[bidx_v],
                            add=True)
            return 0
        lax.fori_loop(0, nbchunk, bbody, 0)

        plsc.subcore_barrier()

        pltpu.sync_copy(
            acc_sh.at[pl.ds(s * zrows_per_tile, zrows_per_tile)],
            deg_out.at[c, pl.ds(s * zrows_per_tile, zrows_per_tile)])

        @pl.when(s == 0)
        def _():
            pltpu.sync_copy(cnt_sh.at[pl.ds(0, nb)], cnt_out.at[c])

    return k(dst_pad, batch_pad)


def _sc_agg(g, src_pad, dst_pad, np_rows, ep):
    rows_per_tile = np_rows // NS
    nchunk = ep // (CH * NC * NS)

    @functools.partial(
        pl.kernel,
        out_type=jax.ShapeDtypeStruct((NC, np_rows, 128), jnp.float32),
        mesh=_mesh(),
        scratch_types=[
            pltpu.VMEM((CH,), jnp.int32),
            pltpu.VMEM((CH,), jnp.int32),
            pltpu.VMEM((CH, 128), jnp.float32),
            pltpu.VMEM((CH, 128), jnp.float32),
            pltpu.VMEM_SHARED((np_rows, 128), jnp.float32),
            pltpu.SemaphoreType.DMA,
        ],
    )
    def k(g_hbm, src_hbm, dst_hbm, out_hbm, sidx_v, didx_v, rows_v, zbuf_v,
          acc_sh, gsem):
        c = lax.axis_index("c")
        s = lax.axis_index("s")
        w = c * NS + s

        def zf(i, _):
            r = i // 8
            q = i % 8
            zbuf_v[r, pl.ds(q * 16, 16)] = jnp.zeros((16,), jnp.float32)
            return 0
        lax.fori_loop(0, CH * 8, zf, 0)

        def zrow(j, _):
            pltpu.sync_copy(zbuf_v, acc_sh.at[pl.ds(s * rows_per_tile
                                                    + j * CH, CH)])
            return 0
        lax.fori_loop(0, rows_per_tile // CH, zrow, 0)
        plsc.subcore_barrier()

        def body(j, _):
            base = (w * nchunk + j) * CH
            pltpu.sync_copy(src_hbm.at[pl.ds(base, CH)], sidx_v)
            pltpu.sync_copy(dst_hbm.at[pl.ds(base, CH)], didx_v)
            pltpu.async_copy(g_hbm.at[sidx_v], rows_v, gsem).wait()
            pltpu.sync_copy(rows_v, acc_sh.at[didx_v], add=True)
            return 0
        lax.fori_loop(0, nchunk, body, 0)
        plsc.subcore_barrier()

        pltpu.sync_copy(
            acc_sh.at[pl.ds(s * rows_per_tile, rows_per_tile)],
            out_hbm.at[c, pl.ds(s * rows_per_tile, rows_per_tile)])

    return k(g, src_pad, dst_pad)


def _sc_pool(h2, batch_pad, np_rows, nb):
    bch = 64
    rows_per_tile = np_rows // (NC * NS)
    nchunk = rows_per_tile // bch
    acc_rows = 2 * nb

    @functools.partial(
        pl.kernel,
        out_type=jax.ShapeDtypeStruct((NC, nb, 128), jnp.float32),
        mesh=_mesh(),
        scratch_types=[
            pltpu.VMEM((bch,), jnp.int32),
            pltpu.VMEM((bch, 128), jnp.float32),
            pltpu.VMEM((CH, 128), jnp.float32),
            pltpu.VMEM_SHARED((2 * nb, 128), jnp.float32),
        ],
    )
    def k(h2_hbm, batch_hbm, out_hbm, bidx_v, rows_v, zbuf_v, acc_sh):
        c = lax.axis_index("c")
        s = lax.axis_index("s")
        w = c * NS + s

        def zf(i, _):
            r = i // 8
            q = i % 8
            zbuf_v[r, pl.ds(q * 16, 16)] = jnp.zeros((16,), jnp.float32)
            return 0
        lax.fori_loop(0, CH * 8, zf, 0)

        @pl.when(s < acc_rows // CH)
        def _():
            pltpu.sync_copy(zbuf_v, acc_sh.at[pl.ds(s * CH, CH)])
        plsc.subcore_barrier()

        def body(j, _):
            base = (w * nchunk + j) * bch
            pltpu.sync_copy(batch_hbm.at[pl.ds(base, bch)], bidx_v)
            pltpu.sync_copy(h2_hbm.at[pl.ds(base, bch)], rows_v)
            pltpu.sync_copy(rows_v, acc_sh.at[bidx_v], add=True)
            return 0
        lax.fori_loop(0, nchunk, body, 0)
        plsc.subcore_barrier()

        @pl.when(s < 8)
        def _():
            pltpu.sync_copy(acc_sh.at[pl.ds(s * 16, 16)],
                            out_hbm.at[c, pl.ds(s * 16, 16)])

    return k(h2, batch_pad)


def kernel(x, edge_index, batch, W1, b1, W2, b2, Wc, bc):
    n, d = x.shape
    e = edge_index.shape[1]
    nb = 128

    runit = CH * NS
    eunit = CH * NC * NS
    np_rows = ((n + runit - 1) // runit) * runit
    ep = ((e + eunit - 1) // eunit) * eunit

    x_pad = jnp.pad(x, ((0, np_rows - n), (0, 0)))
    pad_idx = jnp.full((ep - e,), n, jnp.int32)
    src = jnp.concatenate([edge_index[0], pad_idx])
    dst = jnp.concatenate([edge_index[1], pad_idx])
    batch_pad = jnp.concatenate(
        [batch, jnp.full((np_rows - n,), nb, jnp.int32)])

    b1r = b1.reshape(1, 128)
    b2r = b2.reshape(1, 128)
    bcr = bc.reshape(1, 1)

    h1p = _tc_matmul(x_pad, W1, np_rows)
    deg_st, cnt_st = _sc_deg(dst, batch_pad, np_rows, ep, nb)
    dinv, g1 = _k3(h1p, deg_st, np_rows)
    a1_st = _sc_agg(g1, src, dst, np_rows, ep)
    g2 = _k5(a1_st, g1, dinv, b1r, W2, np_rows)
    a2_st = _sc_agg(g2, src, dst, np_rows, ep)
    h2 = _k7(a2_st, g2, dinv, b2r, np_rows)
    sums_st = _sc_pool(h2, batch_pad, np_rows, nb)
    return _k9(sums_st, cnt_st, Wc, bcr)
